# Initial kernel scaffold; baseline (speedup 1.0000x reference)
import jax
import jax.numpy as jnp
from jax.experimental import pallas as pl


def kernel(features, initial_features, edge_index, W):
    raise NotImplementedError("write your pallas kernel here")


# speedup vs baseline: 3.6420x; 3.6420x over previous
import functools

import jax
import jax.numpy as jnp
from jax import lax
from jax.experimental import pallas as pl
from jax.experimental.pallas import tpu as pltpu
from jax.experimental.pallas import tpu_sc as plsc

ALPHA_C = 0.5
NC = 2
NS = 16
CH = 128
NBUF = 2
SETTLE = 1500


def _settle(buf):
    def body(i, x):
        return x + buf[pl.ds(0, 16)]
    acc = lax.fori_loop(0, SETTLE, body, jnp.zeros((16,), jnp.int32))
    return acc[0]


def _deg_body(npd, rows_per_w, dst2, ones_hbm, zeros_hbm, zi_hbm, degp,
              dstbuf, onesbuf, zibuf, deg_sh):
    c = lax.axis_index("c")
    s = lax.axis_index("s")
    w = c * NS + s
    npt = npd // NS
    del npt
    pltpu.sync_copy(dst2.at[pl.ds(w * rows_per_w, rows_per_w)], dstbuf)
    pltpu.sync_copy(ones_hbm, onesbuf)
    pltpu.sync_copy(zi_hbm, zibuf)

    @pl.when(s == 0)
    def _():
        pltpu.sync_copy(zeros_hbm, deg_sh)
    plsc.subcore_barrier()

    def body(k, carry):
        pltpu.sync_copy(onesbuf, deg_sh.at[dstbuf.at[k]], add=True)
        return carry

    lax.fori_loop(0, rows_per_w, body, None)
    plsc.subcore_barrier()
    v = _settle(zibuf)

    @pl.when(jnp.logical_and(s == 0, v == 0))
    def _():
        pltpu.sync_copy(deg_sh, degp.at[c])


def _deg_kernel(npd, ep, dst2):
    rows_per_w = ep // (NC * NS * CH)
    npt = npd // NS
    ones_hbm = jnp.ones((CH,), jnp.float32)
    zeros_hbm = jnp.zeros((npd,), jnp.float32)
    zi_hbm = jnp.zeros((16,), jnp.int32)
    mesh = plsc.VectorSubcoreMesh(core_axis_name="c", subcore_axis_name="s")
    k = pl.kernel(
        functools.partial(_deg_body, npd, rows_per_w),
        out_type=jax.ShapeDtypeStruct((NC, npd), jnp.float32),
        mesh=mesh,
        scratch_types=[
            pltpu.VMEM((rows_per_w, CH), jnp.int32),
            pltpu.VMEM((CH,), jnp.float32),
            pltpu.VMEM((16,), jnp.int32),
            pltpu.VMEM_SHARED((npd,), jnp.float32),
        ],
    )
    return k(dst2, ones_hbm, zeros_hbm, zi_hbm)


def _scale_body(degp_ref, f_ref, out_ref):
    deg = (degp_ref[0, :] + degp_ref[1, :])[:, None]
    norm = lax.rsqrt(jnp.maximum(deg, 1.0))
    h = f_ref[...] * norm
    out_ref[0] = h[:, :128]
    out_ref[1] = h[:, 128:]


def _scale_kernel(n, npd, d, bn, features, degp):
    grid = ((n + bn - 1) // bn,)
    out = pl.pallas_call(
        _scale_body,
        grid=grid,
        in_specs=[
            pl.BlockSpec((2, bn), lambda i: (0, i)),
            pl.BlockSpec((bn, d), lambda i: (i, 0)),
        ],
        out_specs=pl.BlockSpec((2, bn, 128), lambda i: (0, i, 0)),
        out_shape=jax.ShapeDtypeStruct((2, npd, 128), jnp.float32),
        compiler_params=pltpu.CompilerParams(
            dimension_semantics=("parallel",)),
    )(degp.reshape(2, npd), features)
    return out.reshape(2 * npd, 128)


def _agg_body(npd, ep, kchunks, hs2, gidx2, dst2, zeros_hbm, zi_hbm, agg_hbm,
              agg, gixbuf, dstchk, rowbuf, zibuf, *sems):
    semg = sems[:NBUF]
    semd = sems[NBUF:]
    c = lax.axis_index("c")
    s = lax.axis_index("s")
    npt = npd // NS
    ept = kchunks * CH
    pltpu.sync_copy(gidx2.at[pl.ds(c * ep + s * ept, ept)], gixbuf)
    pltpu.sync_copy(zi_hbm, zibuf)
    pltpu.sync_copy(zeros_hbm, agg.at[pl.ds(s * npt, npt)])

    def issue(b, k):
        pltpu.async_copy(dst2.at[s * kchunks + k], dstchk.at[b], semd[b])
        pltpu.async_copy(hs2.at[gixbuf.at[pl.ds(k * CH, CH)]], rowbuf.at[b],
                         semg[b])

    def wait_and_scatter(b, k):
        pltpu.make_async_copy(dst2.at[s * kchunks + k], dstchk.at[b],
                              semd[b]).wait()
        pltpu.make_async_copy(hs2.at[gixbuf.at[pl.ds(k * CH, CH)]],
                              rowbuf.at[b], semg[b]).wait()
        pltpu.sync_copy(rowbuf.at[b], agg.at[dstchk.at[b]], add=True)

    for b in range(NBUF):
        issue(b, b)
    plsc.subcore_barrier()

    def body(g, carry):
        for b in range(NBUF):
            k = g * NBUF + b
            wait_and_scatter(b, k)
            issue(b, k + NBUF)
        return carry

    lax.fori_loop(0, kchunks // NBUF - 1, body, None)
    for b in range(NBUF):
        wait_and_scatter(b, kchunks - NBUF + b)
    plsc.subcore_barrier()
    v = _settle(zibuf)

    @pl.when(v == 0)
    def _():
        pltpu.sync_copy(agg.at[pl.ds(s * npt, npt)],
                        agg_hbm.at[pl.ds(c * npd + s * npt, npt)])


def _agg_kernel(npd, ep, hs2, gidx2, dst2):
    kchunks = ep // (NS * CH)
    npt = npd // NS
    zeros_hbm = jnp.zeros((npt, 128), jnp.float32)
    zi_hbm = jnp.zeros((16,), jnp.int32)
    mesh = plsc.VectorSubcoreMesh(core_axis_name="c", subcore_axis_name="s")
    k = pl.kernel(
        functools.partial(_agg_body, npd, ep, kchunks),
        out_type=jax.ShapeDtypeStruct((NC * npd, 128), jnp.float32),
        mesh=mesh,
        scratch_types=[
            pltpu.VMEM_SHARED((npd, 128), jnp.float32),
            pltpu.VMEM((kchunks * CH,), jnp.int32),
            pltpu.VMEM((NBUF, CH), jnp.int32),
            pltpu.VMEM((NBUF, CH, 128), jnp.float32),
            pltpu.VMEM((16,), jnp.int32),
        ] + [pltpu.SemaphoreType.DMA] * (2 * NBUF),
    )
    return k(hs2, gidx2, dst2, zeros_hbm, zi_hbm)


def _comb_body(degp_ref, agg_ref, f_ref, ini_ref, w_ref, out_ref):
    deg = (degp_ref[0, :] + degp_ref[1, :])[:, None]
    norm = lax.rsqrt(jnp.maximum(deg, 1.0))
    agg = jnp.concatenate([agg_ref[0], agg_ref[1]], axis=1)
    h = (ALPHA_C * (agg * norm) + ALPHA_C * (ini_ref[...] * (norm * norm))
         + (1.0 - ALPHA_C) * f_ref[...])
    out_ref[...] = lax.dot_general(h, w_ref[...], (((1,), (1,)), ((), ())),
                                   preferred_element_type=jnp.float32)


def _comb_kernel(n, npd, d, bn, degp, agg2, features, initial_features, W):
    grid = ((n + bn - 1) // bn,)
    return pl.pallas_call(
        _comb_body,
        grid=grid,
        in_specs=[
            pl.BlockSpec((2, bn), lambda i: (0, i)),
            pl.BlockSpec((2, bn, 128), lambda i: (0, i, 0)),
            pl.BlockSpec((bn, d), lambda i: (i, 0)),
            pl.BlockSpec((bn, d), lambda i: (i, 0)),
            pl.BlockSpec((d, d), lambda i: (0, 0)),
        ],
        out_specs=pl.BlockSpec((bn, d), lambda i: (i, 0)),
        out_shape=jax.ShapeDtypeStruct((n, d), jnp.float32),
        compiler_params=pltpu.CompilerParams(
            dimension_semantics=("parallel",)),
    )(degp.reshape(2, npd), agg2.reshape(2, npd, 128), features,
      initial_features, W)


def _round_up(x, m):
    return (x + m - 1) // m * m


def kernel(features, initial_features, edge_index, W):
    n, d = features.shape
    e = edge_index.shape[1]
    npd = _round_up(n, NS * 8)
    ep = _round_up(e, NC * NS * CH * 8)
    bn = 2048
    assert d == 256
    assert (ep // (NS * CH)) % NBUF == 0
    src = edge_index[0]
    dst = edge_index[1]
    pad = ep - e
    src_p = jnp.concatenate([src, jnp.full((pad,), npd - 1, jnp.int32)])
    dst_p = jnp.concatenate([dst, jnp.full((pad,), npd - 1, jnp.int32)])
    dst2 = dst_p.reshape(ep // CH, CH)
    gidx2 = jnp.concatenate([src_p, src_p + npd])

    degp = _deg_kernel(npd, ep, dst2)
    hs2 = _scale_kernel(n, npd, d, bn, features, degp)
    agg2 = _agg_kernel(npd, ep, hs2, gidx2, dst2)
    return _comb_kernel(n, npd, d, bn, degp, agg2,
                        features, initial_features, W)
